# Initial kernel scaffold; baseline (speedup 1.0000x reference)
import jax
import jax.numpy as jnp
from jax.experimental import pallas as pl


def kernel(inputs, embeddings):
    raise NotImplementedError("write your pallas kernel here")


# speedup vs baseline: 1.5735x; 1.5735x over previous
import functools

import jax
import jax.numpy as jnp
from jax import lax
from jax.experimental import pallas as pl
from jax.experimental.pallas import tpu as pltpu
from jax.experimental.pallas import tpu_sc as plsc

VOCAB = 1000000
EMBED = 32
BATCH = 16384
FIELDS = 26

NW = 32
TOTAL = BATCH * FIELDS
RPW = TOTAL // NW
GW = 128
NG = RPW // GW
PER_CHUNK = 13
CHUNK = PER_CHUNK * GW
NCHUNK = NG // PER_CHUNK

_mesh = plsc.VectorSubcoreMesh(core_axis_name="c", subcore_axis_name="s")


@functools.partial(
    pl.kernel,
    mesh=_mesh,
    compiler_params=pltpu.CompilerParams(use_tc_tiling_on_sc=False),
    out_type=jax.ShapeDtypeStruct((TOTAL, EMBED), jnp.float32),
    scratch_types=[
        pltpu.VMEM((NG, GW), jnp.int32),
        pltpu.VMEM((CHUNK, EMBED), jnp.float32),
        pltpu.VMEM((CHUNK, EMBED), jnp.float32),
        pltpu.SemaphoreType.DMA,
        pltpu.SemaphoreType.DMA,
    ],
)
def _embed_gather(idx_hbm, table_hbm, out_hbm, idx_v, rows_a, rows_b,
                  sem_a, sem_b):
  wid = lax.axis_index("s") * 2 + lax.axis_index("c")
  base = wid * RPW

  pltpu.sync_copy(idx_hbm.at[wid], idx_v)

  def fire(c, buf, sem):
    for j in range(PER_CHUNK):
      pltpu.async_copy(
          table_hbm.at[idx_v.at[c * PER_CHUNK + j]],
          buf.at[pl.ds(j * GW, GW)],
          sem,
      )

  def drain(buf, sem):
    pltpu.make_async_copy(table_hbm.at[pl.ds(0, CHUNK)], buf, sem).wait()

  def store(c, buf):
    pltpu.sync_copy(buf, out_hbm.at[pl.ds(base + c * CHUNK, CHUNK)])

  fire(0, rows_a, sem_a)

  def body(t, carry):
    c0 = 2 * t
    fire(c0 + 1, rows_b, sem_b)
    drain(rows_a, sem_a)
    store(c0, rows_a)

    @pl.when(t < NCHUNK // 2 - 1)
    def _():
      fire(c0 + 2, rows_a, sem_a)

    drain(rows_b, sem_b)
    store(c0 + 1, rows_b)
    return carry

  lax.fori_loop(0, NCHUNK // 2, body, 0)


def kernel(inputs, embeddings):
  idx = inputs.astype(jnp.int32).reshape(NW, NG, GW)
  out = _embed_gather(idx, embeddings)
  return out.reshape(BATCH, FIELDS, EMBED)


# speedup vs baseline: 1.5817x; 1.0052x over previous
import functools

import jax
import jax.numpy as jnp
from jax import lax
from jax.experimental import pallas as pl
from jax.experimental.pallas import tpu as pltpu
from jax.experimental.pallas import tpu_sc as plsc

VOCAB = 1000000
EMBED = 32
BATCH = 16384
FIELDS = 26

NW = 32
TOTAL = BATCH * FIELDS
RPW = TOTAL // NW
GW = 1664
NG = RPW // GW
PER_CHUNK = 1
CHUNK = PER_CHUNK * GW
NCHUNK = NG // PER_CHUNK

_mesh = plsc.VectorSubcoreMesh(core_axis_name="c", subcore_axis_name="s")


@functools.partial(
    pl.kernel,
    mesh=_mesh,
    compiler_params=pltpu.CompilerParams(use_tc_tiling_on_sc=False),
    out_type=jax.ShapeDtypeStruct((TOTAL, EMBED), jnp.float32),
    scratch_types=[
        pltpu.VMEM((NG, GW), jnp.int32),
        pltpu.VMEM((CHUNK, EMBED), jnp.float32),
        pltpu.VMEM((CHUNK, EMBED), jnp.float32),
        pltpu.SemaphoreType.DMA,
        pltpu.SemaphoreType.DMA,
    ],
)
def _embed_gather(idx_hbm, table_hbm, out_hbm, idx_v, rows_a, rows_b,
                  sem_a, sem_b):
  wid = lax.axis_index("s") * 2 + lax.axis_index("c")
  base = wid * RPW

  pltpu.sync_copy(idx_hbm.at[wid], idx_v)

  def fire(c, buf, sem):
    for j in range(PER_CHUNK):
      pltpu.async_copy(
          table_hbm.at[idx_v.at[c * PER_CHUNK + j]],
          buf.at[pl.ds(j * GW, GW)],
          sem,
      )

  def drain(buf, sem):
    pltpu.make_async_copy(table_hbm.at[pl.ds(0, CHUNK)], buf, sem).wait()

  def store(c, buf):
    pltpu.sync_copy(buf, out_hbm.at[pl.ds(base + c * CHUNK, CHUNK)])

  fire(0, rows_a, sem_a)

  def body(t, carry):
    c0 = 2 * t
    fire(c0 + 1, rows_b, sem_b)
    drain(rows_a, sem_a)
    store(c0, rows_a)

    @pl.when(t < NCHUNK // 2 - 1)
    def _():
      fire(c0 + 2, rows_a, sem_a)

    drain(rows_b, sem_b)
    store(c0 + 1, rows_b)
    return carry

  lax.fori_loop(0, NCHUNK // 2, body, 0)


def kernel(inputs, embeddings):
  idx = inputs.astype(jnp.int32).reshape(NW, NG, GW)
  out = _embed_gather(idx, embeddings)
  return out.reshape(BATCH, FIELDS, EMBED)


# speedup vs baseline: 1.6725x; 1.0574x over previous
import functools

import jax
import jax.numpy as jnp
from jax import lax
from jax.experimental import pallas as pl
from jax.experimental.pallas import tpu as pltpu
from jax.experimental.pallas import tpu_sc as plsc

VOCAB = 1000000
EMBED = 32
BATCH = 16384
FIELDS = 26

NW = 32
TOTAL = BATCH * FIELDS
RPW = TOTAL // NW
GW = 1664
NG = RPW // GW
PER_CHUNK = 1
CHUNK = PER_CHUNK * GW
NCHUNK = NG // PER_CHUNK

_mesh = plsc.VectorSubcoreMesh(core_axis_name="c", subcore_axis_name="s")


@functools.partial(
    pl.kernel,
    mesh=_mesh,
    compiler_params=pltpu.CompilerParams(use_tc_tiling_on_sc=False),
    out_type=jax.ShapeDtypeStruct((TOTAL, EMBED), jnp.float32),
    scratch_types=[
        pltpu.VMEM((RPW,), jnp.int32),
        pltpu.VMEM((CHUNK, EMBED), jnp.float32),
        pltpu.VMEM((CHUNK, EMBED), jnp.float32),
        pltpu.SemaphoreType.DMA,
        pltpu.SemaphoreType.DMA,
    ],
)
def _embed_gather(idx_hbm, table_hbm, out_hbm, idx_v, rows_a, rows_b,
                  sem_a, sem_b):
  wid = lax.axis_index("s") * 2 + lax.axis_index("c")
  base = wid * RPW

  pltpu.sync_copy(idx_hbm.at[pl.ds(base, RPW)], idx_v)

  def fire(c, buf, sem):
    for j in range(PER_CHUNK):
      pltpu.async_copy(
          table_hbm.at[idx_v.at[pl.ds((c * PER_CHUNK + j) * GW, GW)]],
          buf.at[pl.ds(j * GW, GW)],
          sem,
      )

  def drain(buf, sem):
    pltpu.make_async_copy(table_hbm.at[pl.ds(0, CHUNK)], buf, sem).wait()

  def store(c, buf):
    pltpu.sync_copy(buf, out_hbm.at[pl.ds(base + c * CHUNK, CHUNK)])

  fire(0, rows_a, sem_a)

  def body(t, carry):
    c0 = 2 * t
    fire(c0 + 1, rows_b, sem_b)
    drain(rows_a, sem_a)
    store(c0, rows_a)

    @pl.when(t < NCHUNK // 2 - 1)
    def _():
      fire(c0 + 2, rows_a, sem_a)

    drain(rows_b, sem_b)
    store(c0 + 1, rows_b)
    return carry

  lax.fori_loop(0, NCHUNK // 2, body, 0)


def kernel(inputs, embeddings):
  idx = jnp.swapaxes(inputs, 0, 1).reshape(TOTAL).astype(jnp.int32)
  out = _embed_gather(idx, embeddings)
  return out.reshape(FIELDS, BATCH, EMBED).transpose(1, 0, 2)
